# baseline (speedup 1.0000x reference)
import functools

import jax
import jax.numpy as jnp
from jax import lax
from jax.experimental import pallas as pl
from jax.experimental.pallas import tpu as pltpu
from jax.experimental.pallas import tpu_sc as plsc

NV = 10000
NE = 320000
NC = 2
NS = 16
VP = 10240
ROWS_PT = VP // NS
EB = 80
_EPS = 1e-5

def _vmesh():
    return plsc.VectorSubcoreMesh(core_axis_name="c", subcore_axis_name="s",
                                  num_cores=NC, num_subcores=NS)


_SC_PARAMS = pltpu.CompilerParams(use_tc_tiling_on_sc=False)


def _sc_degree(dstr, ones, zeros):
    nbuf = 5
    dch = NE // (NC * NS * EB)

    @functools.partial(
        pl.kernel,
        out_type=jax.ShapeDtypeStruct((NC, VP, 16), jnp.float32),
        mesh=_vmesh(),
        compiler_params=_SC_PARAMS,
        scratch_types=[
            pltpu.VMEM_SHARED((VP, 16), jnp.float32),
            pltpu.VMEM((dch, EB), jnp.int32),
            pltpu.VMEM((EB, 16), jnp.float32),
        ] + [pltpu.SemaphoreType.DMA] * nbuf,
    )
    def k(dst_hbm, ones_hbm, zeros_hbm, out_hbm, acc, idxb, ones_v, *sems):
        c = lax.axis_index("c")
        s = lax.axis_index("s")
        r0 = s * ROWS_PT
        pltpu.sync_copy(zeros_hbm.at[pl.ds(r0, ROWS_PT)],
                        acc.at[pl.ds(r0, ROWS_PT)])
        pltpu.sync_copy(dst_hbm.at[c * NS + s], idxb)
        pltpu.sync_copy(ones_hbm, ones_v)
        plsc.subcore_barrier()

        @pl.loop(0, dch // nbuf)
        def _(it):
            base = it * nbuf
            ds_ = [pltpu.async_copy(ones_v, acc.at[idxb.at[base + j]],
                                    sems[j], add=True) for j in range(nbuf)]
            for d in ds_:
                d.wait()

        plsc.subcore_barrier()
        pltpu.sync_copy(acc.at[pl.ds(r0, ROWS_PT)],
                        out_hbm.at[c, pl.ds(r0, ROWS_PT)])

    return k(dstr, ones, zeros)


def _sc_prop(zs, srcadj, dstr, zeros, c2):
    nbuf = 10
    pch = NE // (NS * EB)
    sbr = 5
    sbc = sbr * nbuf
    nsb = pch // sbc

    @functools.partial(
        pl.kernel,
        out_type=jax.ShapeDtypeStruct((NC * VP, c2), jnp.float32),
        mesh=_vmesh(),
        compiler_params=_SC_PARAMS,
        scratch_types=[
            pltpu.VMEM_SHARED((VP, c2), jnp.float32),
            pltpu.VMEM((sbc, EB), jnp.int32),
            pltpu.VMEM((sbc, EB), jnp.int32),
        ] + [pltpu.VMEM((EB, c2), jnp.float32)] * nbuf
          + [pltpu.SemaphoreType.DMA] * (2 * nbuf),
    )
    def k(z_hbm, src_hbm, dst_hbm, zeros_hbm, out_hbm, acc, igat, idst, *rest):
        rows = rest[:nbuf]
        gsem = rest[nbuf:2 * nbuf]
        ssem = rest[2 * nbuf:]
        c = lax.axis_index("c")
        s = lax.axis_index("s")
        r0 = s * ROWS_PT
        pltpu.sync_copy(zeros_hbm.at[pl.ds(r0, ROWS_PT)],
                        acc.at[pl.ds(r0, ROWS_PT)])
        plsc.subcore_barrier()

        @pl.loop(0, nsb)
        def _(sb):
            base = sb * sbc
            pltpu.sync_copy(src_hbm.at[c, s, pl.ds(base, sbc)], igat)
            pltpu.sync_copy(dst_hbm.at[s, pl.ds(base, sbc)], idst)
            gds = [pltpu.async_copy(z_hbm.at[igat.at[j]], rows[j], gsem[j])
                   for j in range(nbuf)]
            for r in range(sbr):
                sds = []
                for j in range(nbuf):
                    gds[j].wait()
                    sds.append(pltpu.async_copy(
                        rows[j], acc.at[idst.at[r * nbuf + j]], ssem[j],
                        add=True))
                if r + 1 < sbr:
                    gds = []
                    for j in range(nbuf):
                        sds[j].wait()
                        gds.append(pltpu.async_copy(
                            z_hbm.at[igat.at[(r + 1) * nbuf + j]], rows[j],
                            gsem[j]))
                else:
                    for d in sds:
                        d.wait()

        plsc.subcore_barrier()
        pltpu.sync_copy(acc.at[pl.ds(r0, ROWS_PT)],
                        out_hbm.at[pl.ds(c * VP + r0, ROWS_PT)])

    return k(zs, srcadj, dstr, zeros)


def _tc_prep(degp, x):
    def body(degp_ref, x_ref, dis_ref, z_ref):
        deg = degp_ref[0, 0:NV] + degp_ref[1, 0:NV]
        dis = jnp.where(deg > 0.0, lax.rsqrt(jnp.maximum(deg, 1e-12)), 0.0)
        dis_ref[...] = dis
        z = dis[:, 0:1] * x_ref[...]
        z_ref[0:NV] = z[:, 0:64]
        z_ref[VP:VP + NV] = z[:, 64:128]

    return pl.pallas_call(
        body,
        out_shape=(jax.ShapeDtypeStruct((NV, 16), jnp.float32),
                   jax.ShapeDtypeStruct((2 * VP, 64), jnp.float32)),
    )(degp, x)


def _tc_a(h, w, ci, co):
    rb = 1000
    nb = NV // rb

    def body(h_ref, w_ref, o_ref):
        o_ref[...] = jnp.dot(h_ref[...], w_ref[...],
                             preferred_element_type=jnp.float32)

    return pl.pallas_call(
        body,
        grid=(nb,),
        in_specs=[pl.BlockSpec((rb, ci), lambda i: (i, 0)),
                  pl.BlockSpec((ci, co), lambda i: (0, 0))],
        out_specs=pl.BlockSpec((rb, co), lambda i: (i, 0)),
        out_shape=jax.ShapeDtypeStruct((NV, co), jnp.float32),
    )(h, w)


def _tc_mid(p1r, dis16, ci):
    c2i = ci // 2
    rb = 1000
    nb = NV // rb

    def body(p1_ref, dis_ref, u_ref, z2_ref):
        d1 = dis_ref[:, 0:1]
        u = -d1 * jnp.concatenate([p1_ref[0], p1_ref[1]], axis=1)
        u_ref[...] = u
        z2 = d1 * u
        z2_ref[0] = z2[:, 0:c2i]
        z2_ref[1] = z2[:, c2i:]

    return pl.pallas_call(
        body,
        grid=(nb,),
        in_specs=[pl.BlockSpec((2, rb, c2i), lambda i: (0, i, 0)),
                  pl.BlockSpec((rb, 16), lambda i: (i, 0))],
        out_specs=(pl.BlockSpec((rb, ci), lambda i: (i, 0)),
                   pl.BlockSpec((2, rb, c2i), lambda i: (0, i, 0))),
        out_shape=(jax.ShapeDtypeStruct((NV, ci), jnp.float32),
                   jax.ShapeDtypeStruct((2, VP, c2i), jnp.float32)),
    )(p1r, dis16)


def _tc_fin(a0, a1, p2r, h, dis16, W, b, g, be, ci, co, emit_z):
    c2i = ci // 2
    c2 = co // 2
    gs = co // 8
    rb = 1000
    nb = NV // rb

    def body(a0_ref, a1_ref, p2_ref, h_ref, dis_ref, w_ref, b_ref, g_ref,
             be_ref, *outs):
        d1 = dis_ref[:, 0:1]
        tx2 = (-2.0 * d1) * jnp.concatenate([p2_ref[0], p2_ref[1]], axis=1) \
            - h_ref[...]
        t = a0_ref[...] + a1_ref[...] \
            + jnp.dot(tx2, w_ref[2], preferred_element_type=jnp.float32) \
            + b_ref[...]
        gm = (lax.broadcasted_iota(jnp.int32, (co, 8), 0) // gs
              == lax.broadcasted_iota(jnp.int32, (co, 8), 1)
              ).astype(jnp.float32)
        gmt = (lax.broadcasted_iota(jnp.int32, (8, co), 0)
               == lax.broadcasted_iota(jnp.int32, (8, co), 1) // gs
               ).astype(jnp.float32)
        hp = lax.Precision.HIGHEST
        mu = jnp.dot(jnp.dot(t, gm, preferred_element_type=jnp.float32,
                             precision=hp) * (1.0 / gs),
                     gmt, preferred_element_type=jnp.float32, precision=hp)
        dv = t - mu
        var = jnp.dot(jnp.dot(dv * dv, gm, preferred_element_type=jnp.float32,
                              precision=hp)
                      * (1.0 / gs), gmt, preferred_element_type=jnp.float32,
                      precision=hp)
        hn = dv * lax.rsqrt(var + _EPS) * g_ref[...] + be_ref[...]
        hh = jnp.maximum(hn, 0.0)
        outs[0][...] = hh
        if emit_z:
            z = d1 * hh
            outs[1][0] = z[:, 0:c2]
            outs[1][1] = z[:, c2:]

    out_shape = [jax.ShapeDtypeStruct((NV, co), jnp.float32)]
    out_specs = [pl.BlockSpec((rb, co), lambda i: (i, 0))]
    if emit_z:
        out_shape.append(jax.ShapeDtypeStruct((2, VP, c2), jnp.float32))
        out_specs.append(pl.BlockSpec((2, rb, c2), lambda i: (0, i, 0)))
    return pl.pallas_call(
        body,
        grid=(nb,),
        in_specs=[pl.BlockSpec((rb, co), lambda i: (i, 0)),
                  pl.BlockSpec((rb, co), lambda i: (i, 0)),
                  pl.BlockSpec((2, rb, c2i), lambda i: (0, i, 0)),
                  pl.BlockSpec((rb, ci), lambda i: (i, 0)),
                  pl.BlockSpec((rb, 16), lambda i: (i, 0)),
                  pl.BlockSpec((3, ci, co), lambda i: (0, 0, 0)),
                  pl.BlockSpec((1, co), lambda i: (0, 0)),
                  pl.BlockSpec((1, co), lambda i: (0, 0)),
                  pl.BlockSpec((1, co), lambda i: (0, 0))],
        out_specs=tuple(out_specs),
        out_shape=tuple(out_shape),
    )(a0, a1, p2r, h, dis16, W, b.reshape(1, co), g.reshape(1, co),
      be.reshape(1, co))


def _tc_linear(hflat, Wl, bl):
    nblk = 10
    kb = (NV * 32) // nblk

    def body(w_ref, h_ref, b_ref, o_ref):
        i = pl.program_id(0)

        @pl.when(i == 0)
        def _():
            o_ref[...] = b_ref[...]

        hb = h_ref[:, pl.ds(i * kb, kb)]
        o_ref[...] += lax.dot_general(
            hb, w_ref[...], (((1,), (1,)), ((), ())),
            preferred_element_type=jnp.float32)

    return pl.pallas_call(
        body,
        grid=(nblk,),
        in_specs=[pl.BlockSpec((32, kb), lambda i: (0, i)),
                  pl.BlockSpec((1, NV * 32), lambda i: (0, 0)),
                  pl.BlockSpec((1, 32), lambda i: (0, 0))],
        out_specs=pl.BlockSpec((1, 32), lambda i: (0, 0)),
        out_shape=jax.ShapeDtypeStruct((1, 32), jnp.float32),
    )(Wl, hflat, bl.reshape(1, 32))


def kernel(x, edge_index, W1, b1, g1, be1, W2, b2, g2, be2, W3, b3, g3, be3,
           Wl, bl):
    src = edge_index[0]
    dst = edge_index[1]
    pch = NE // (NS * EB)
    dch = NE // (NC * NS * EB)
    srcadj = jnp.stack([src, src + VP]).reshape(NC, NS, pch, EB)
    dstr = dst.reshape(NS, pch, EB)
    dstd = dst.reshape(NC * NS, dch, EB)

    degp = _sc_degree(dstd, jnp.ones((EB, 16), jnp.float32),
                      jnp.zeros((VP, 16), jnp.float32))
    dis16, z = _tc_prep(degp, x)

    h = x
    dims = [(128, 128), (128, 64), (64, 32)]
    for (ci, co), W, b, g, be in zip(dims, (W1, W2, W3), (b1, b2, b3),
                                     (g1, g2, g3), (be1, be2, be3)):
        c2i = ci // 2
        zeros_i = jnp.zeros((VP, c2i), jnp.float32)
        a0 = _tc_a(h, W[0], ci, co)
        p1 = _sc_prop(z, srcadj, dstr, zeros_i, c2i)
        u, z2 = _tc_mid(p1.reshape(2, VP, c2i), dis16, ci)
        a1 = _tc_a(u, W[1], ci, co)
        p2 = _sc_prop(z2.reshape(2 * VP, c2i), srcadj, dstr, zeros_i, c2i)
        last = co == 32
        outs = _tc_fin(a0, a1, p2.reshape(2, VP, c2i), h, dis16, W, b, g, be,
                       ci, co, emit_z=not last)
        h = outs[0]
        if not last:
            z = outs[1].reshape(2 * VP, co // 2)

    return _tc_linear(h.reshape(1, NV * 32), Wl, bl)


# speedup vs baseline: 1.0948x; 1.0948x over previous
import functools

import jax
import jax.numpy as jnp
from jax import lax
from jax.experimental import pallas as pl
from jax.experimental.pallas import tpu as pltpu
from jax.experimental.pallas import tpu_sc as plsc

NV = 10000
NE = 320000
NC = 2
NS = 16
VP = 10240
ROWS_PT = VP // NS
EB = 80
_EPS = 1e-5

def _vmesh():
    return plsc.VectorSubcoreMesh(core_axis_name="c", subcore_axis_name="s",
                                  num_cores=NC, num_subcores=NS)


_SC_PARAMS = pltpu.CompilerParams(use_tc_tiling_on_sc=False)


def _sc_degree(dstr, ones, zeros):
    nbuf = 5
    dch = NE // (NC * NS * EB)

    @functools.partial(
        pl.kernel,
        out_type=jax.ShapeDtypeStruct((NC, VP, 16), jnp.float32),
        mesh=_vmesh(),
        compiler_params=_SC_PARAMS,
        scratch_types=[
            pltpu.VMEM_SHARED((VP, 16), jnp.float32),
            pltpu.VMEM((dch, EB), jnp.int32),
            pltpu.VMEM((EB, 16), jnp.float32),
        ] + [pltpu.SemaphoreType.DMA] * nbuf,
    )
    def k(dst_hbm, ones_hbm, zeros_hbm, out_hbm, acc, idxb, ones_v, *sems):
        c = lax.axis_index("c")
        s = lax.axis_index("s")
        r0 = s * ROWS_PT
        pltpu.sync_copy(zeros_hbm.at[pl.ds(r0, ROWS_PT)],
                        acc.at[pl.ds(r0, ROWS_PT)])
        pltpu.sync_copy(dst_hbm.at[c * NS + s], idxb)
        pltpu.sync_copy(ones_hbm, ones_v)
        plsc.subcore_barrier()

        @pl.loop(0, dch // nbuf)
        def _(it):
            base = it * nbuf
            ds_ = [pltpu.async_copy(ones_v, acc.at[idxb.at[base + j]],
                                    sems[j], add=True) for j in range(nbuf)]
            for d in ds_:
                d.wait()

        plsc.subcore_barrier()
        pltpu.sync_copy(acc.at[pl.ds(r0, ROWS_PT)],
                        out_hbm.at[c, pl.ds(r0, ROWS_PT)])

    return k(dstr, ones, zeros)


def _sc_prop(zs, srcr, dstr, zeros, c2):
    nbuf = 10
    pch = NE // (NS * EB)
    sbr = 5
    sbc = sbr * nbuf
    nsb = pch // sbc

    @functools.partial(
        pl.kernel,
        out_type=jax.ShapeDtypeStruct((NC, VP, c2), jnp.float32),
        mesh=_vmesh(),
        compiler_params=_SC_PARAMS,
        scratch_types=[
            pltpu.VMEM_SHARED((VP, c2), jnp.float32),
            pltpu.VMEM((sbc, EB), jnp.int32),
            pltpu.VMEM((sbc, EB), jnp.int32),
        ] + [pltpu.VMEM((EB, c2), jnp.float32)] * nbuf
          + [pltpu.SemaphoreType.DMA] * (2 * nbuf),
    )
    def k(z_hbm, src_hbm, dst_hbm, zeros_hbm, out_hbm, acc, igat, idst, *rest):
        rows = rest[:nbuf]
        gsem = rest[nbuf:2 * nbuf]
        ssem = rest[2 * nbuf:]
        c = lax.axis_index("c")
        s = lax.axis_index("s")
        zc = z_hbm.at[c]
        r0 = s * ROWS_PT
        pltpu.sync_copy(zeros_hbm.at[pl.ds(r0, ROWS_PT)],
                        acc.at[pl.ds(r0, ROWS_PT)])
        plsc.subcore_barrier()

        @pl.loop(0, nsb)
        def _(sb):
            base = sb * sbc
            pltpu.sync_copy(src_hbm.at[s, pl.ds(base, sbc)], igat)
            pltpu.sync_copy(dst_hbm.at[s, pl.ds(base, sbc)], idst)
            gds = [pltpu.async_copy(zc.at[igat.at[j]], rows[j], gsem[j])
                   for j in range(nbuf)]
            for r in range(sbr):
                sds = []
                for j in range(nbuf):
                    gds[j].wait()
                    sds.append(pltpu.async_copy(
                        rows[j], acc.at[idst.at[r * nbuf + j]], ssem[j],
                        add=True))
                if r + 1 < sbr:
                    gds = []
                    for j in range(nbuf):
                        sds[j].wait()
                        gds.append(pltpu.async_copy(
                            zc.at[igat.at[(r + 1) * nbuf + j]], rows[j],
                            gsem[j]))
                else:
                    for d in sds:
                        d.wait()

        plsc.subcore_barrier()
        pltpu.sync_copy(acc.at[pl.ds(r0, ROWS_PT)],
                        out_hbm.at[c, pl.ds(r0, ROWS_PT)])

    return k(zs, srcr, dstr, zeros)


def _tc_prep(degp, x):
    def body(degp_ref, x_ref, dis_ref, z_ref):
        deg = degp_ref[0, 0:NV] + degp_ref[1, 0:NV]
        dis = jnp.where(deg > 0.0, lax.rsqrt(jnp.maximum(deg, 1e-12)), 0.0)
        dis_ref[...] = dis
        z = dis[:, 0:1] * x_ref[...]
        z_ref[0, 0:NV] = z[:, 0:64]
        z_ref[1, 0:NV] = z[:, 64:128]

    return pl.pallas_call(
        body,
        out_shape=(jax.ShapeDtypeStruct((NV, 16), jnp.float32),
                   jax.ShapeDtypeStruct((2, VP, 64), jnp.float32)),
    )(degp, x)


def _tc_a(h, w, ci, co):
    rb = 1000
    nb = NV // rb

    def body(h_ref, w_ref, o_ref):
        o_ref[...] = jnp.dot(h_ref[...], w_ref[...],
                             preferred_element_type=jnp.float32)

    return pl.pallas_call(
        body,
        grid=(nb,),
        in_specs=[pl.BlockSpec((rb, ci), lambda i: (i, 0)),
                  pl.BlockSpec((ci, co), lambda i: (0, 0))],
        out_specs=pl.BlockSpec((rb, co), lambda i: (i, 0)),
        out_shape=jax.ShapeDtypeStruct((NV, co), jnp.float32),
    )(h, w)


def _tc_mid(p1r, dis16, ci):
    c2i = ci // 2
    rb = 1000
    nb = NV // rb

    def body(p1_ref, dis_ref, u_ref, z2_ref):
        d1 = dis_ref[:, 0:1]
        u = -d1 * jnp.concatenate([p1_ref[0], p1_ref[1]], axis=1)
        u_ref[...] = u
        z2 = d1 * u
        z2_ref[0] = z2[:, 0:c2i]
        z2_ref[1] = z2[:, c2i:]

    return pl.pallas_call(
        body,
        grid=(nb,),
        in_specs=[pl.BlockSpec((2, rb, c2i), lambda i: (0, i, 0)),
                  pl.BlockSpec((rb, 16), lambda i: (i, 0))],
        out_specs=(pl.BlockSpec((rb, ci), lambda i: (i, 0)),
                   pl.BlockSpec((2, rb, c2i), lambda i: (0, i, 0))),
        out_shape=(jax.ShapeDtypeStruct((NV, ci), jnp.float32),
                   jax.ShapeDtypeStruct((2, VP, c2i), jnp.float32)),
    )(p1r, dis16)


def _tc_fin(a0, a1, p2r, h, dis16, W, b, g, be, ci, co, emit_z):
    c2i = ci // 2
    c2 = co // 2
    gs = co // 8
    rb = 1000
    nb = NV // rb

    def body(a0_ref, a1_ref, p2_ref, h_ref, dis_ref, w_ref, b_ref, g_ref,
             be_ref, *outs):
        d1 = dis_ref[:, 0:1]
        tx2 = (-2.0 * d1) * jnp.concatenate([p2_ref[0], p2_ref[1]], axis=1) \
            - h_ref[...]
        t = a0_ref[...] + a1_ref[...] \
            + jnp.dot(tx2, w_ref[2], preferred_element_type=jnp.float32) \
            + b_ref[...]
        m2 = jnp.where(lax.broadcasted_iota(jnp.int32, (co, co), 0) // gs
                       == lax.broadcasted_iota(jnp.int32, (co, co), 1) // gs,
                       1.0 / gs, 0.0)
        hp = lax.Precision.HIGHEST
        mu = jnp.dot(t, m2, preferred_element_type=jnp.float32, precision=hp)
        dv = t - mu
        var = jnp.dot(dv * dv, m2, preferred_element_type=jnp.float32,
                      precision=hp)
        hn = dv * lax.rsqrt(var + _EPS) * g_ref[...] + be_ref[...]
        hh = jnp.maximum(hn, 0.0)
        outs[0][...] = hh
        if emit_z:
            z = d1 * hh
            outs[1][0] = z[:, 0:c2]
            outs[1][1] = z[:, c2:]

    out_shape = [jax.ShapeDtypeStruct((NV, co), jnp.float32)]
    out_specs = [pl.BlockSpec((rb, co), lambda i: (i, 0))]
    if emit_z:
        out_shape.append(jax.ShapeDtypeStruct((2, VP, c2), jnp.float32))
        out_specs.append(pl.BlockSpec((2, rb, c2), lambda i: (0, i, 0)))
    return pl.pallas_call(
        body,
        grid=(nb,),
        in_specs=[pl.BlockSpec((rb, co), lambda i: (i, 0)),
                  pl.BlockSpec((rb, co), lambda i: (i, 0)),
                  pl.BlockSpec((2, rb, c2i), lambda i: (0, i, 0)),
                  pl.BlockSpec((rb, ci), lambda i: (i, 0)),
                  pl.BlockSpec((rb, 16), lambda i: (i, 0)),
                  pl.BlockSpec((3, ci, co), lambda i: (0, 0, 0)),
                  pl.BlockSpec((1, co), lambda i: (0, 0)),
                  pl.BlockSpec((1, co), lambda i: (0, 0)),
                  pl.BlockSpec((1, co), lambda i: (0, 0))],
        out_specs=tuple(out_specs),
        out_shape=tuple(out_shape),
    )(a0, a1, p2r, h, dis16, W, b.reshape(1, co), g.reshape(1, co),
      be.reshape(1, co))


def _tc_linear(hflat, Wl, bl):
    nblk = 10
    kb = (NV * 32) // nblk

    def body(w_ref, h_ref, b_ref, o_ref):
        i = pl.program_id(0)

        @pl.when(i == 0)
        def _():
            o_ref[...] = b_ref[...]

        hb = h_ref[:, pl.ds(i * kb, kb)]
        o_ref[...] += lax.dot_general(
            hb, w_ref[...], (((1,), (1,)), ((), ())),
            preferred_element_type=jnp.float32)

    return pl.pallas_call(
        body,
        grid=(nblk,),
        in_specs=[pl.BlockSpec((32, kb), lambda i: (0, i)),
                  pl.BlockSpec((1, NV * 32), lambda i: (0, 0)),
                  pl.BlockSpec((1, 32), lambda i: (0, 0))],
        out_specs=pl.BlockSpec((1, 32), lambda i: (0, 0)),
        out_shape=jax.ShapeDtypeStruct((1, 32), jnp.float32),
    )(Wl, hflat, bl.reshape(1, 32))


def kernel(x, edge_index, W1, b1, g1, be1, W2, b2, g2, be2, W3, b3, g3, be3,
           Wl, bl):
    src = edge_index[0]
    dst = edge_index[1]
    pch = NE // (NS * EB)
    dch = NE // (NC * NS * EB)
    srcr = src.reshape(NS, pch, EB)
    dstr = dst.reshape(NS, pch, EB)
    dstd = dst.reshape(NC * NS, dch, EB)

    degp = _sc_degree(dstd, jnp.ones((EB, 16), jnp.float32),
                      jnp.zeros((VP, 16), jnp.float32))
    dis16, z = _tc_prep(degp, x)

    h = x
    dims = [(128, 128), (128, 64), (64, 32)]
    for (ci, co), W, b, g, be in zip(dims, (W1, W2, W3), (b1, b2, b3),
                                     (g1, g2, g3), (be1, be2, be3)):
        c2i = ci // 2
        zeros_i = jnp.zeros((VP, c2i), jnp.float32)
        a0 = _tc_a(h, W[0], ci, co)
        p1 = _sc_prop(z, srcr, dstr, zeros_i, c2i)
        u, z2 = _tc_mid(p1, dis16, ci)
        a1 = _tc_a(u, W[1], ci, co)
        p2 = _sc_prop(z2, srcr, dstr, zeros_i, c2i)
        last = co == 32
        outs = _tc_fin(a0, a1, p2, h, dis16, W, b, g, be, ci, co,
                       emit_z=not last)
        h = outs[0]
        if not last:
            z = outs[1]

    return _tc_linear(h.reshape(1, NV * 32), Wl, bl)
